# baseline (speedup 1.0000x reference)
import jax
import jax.numpy as jnp
from jax import lax
from jax.experimental import pallas as pl
from jax.experimental.pallas import tpu as pltpu
from jax.experimental.pallas import tpu_sc as plsc

DIM = 128
GLOVE_DIM = 300
OUT_DIM = DIM + GLOVE_DIM
COMB_DIM = 512

NC = 2
NS = 16
NW = NC * NS
LANES = 16

CB = 2
L = 50
CL = CB * L


def _sc_body(x_hbm, comb_hbm, out_hbm,
             idx2_a, idx2_b, idx_a, idx_b, comb_a, comb_b,
             sem_a, sem_b, sem_wa, sem_wb):
    wid = lax.axis_index("s") * NC + lax.axis_index("c")
    nb_total = x_hbm.shape[0]
    per_w = nb_total // NW
    n_chunks = per_w // CB
    n2 = n_chunks // 2
    base_w = wid * per_w

    def start_gather(c, idx2_v, idx_v, comb_v, sem):
        b0 = base_w + c * CB
        pltpu.sync_copy(x_hbm.at[pl.ds(b0, CB)], idx2_v)

        def flat_body(m, carry2):
            q = jax.lax.iota(jnp.int32, LANES) + m * LANES
            msk = q < CL
            qc = jnp.minimum(q, CL - 1)
            bb = qc // L
            l = qc - bb * L
            vals = plsc.load_gather(idx2_v, [bb, l], mask=msk)
            plsc.store_scatter(idx_v, [qc], vals, mask=msk)
            return carry2

        lax.fori_loop(0, (CL + LANES - 1) // LANES, flat_body, 0, unroll=2)
        pltpu.async_copy(comb_hbm.at[idx_v], comb_v, sem)

    def wait_gather(idx_v, comb_v, sem):
        pltpu.make_async_copy(comb_hbm.at[idx_v], comb_v, sem).wait()

    def start_write(c, comb_v, sem):
        r0 = (base_w + c * CB) * L
        pltpu.async_copy(comb_v, out_hbm.at[pl.ds(r0, CL)], sem)

    def drain_write(c, comb_v, sem):
        r0 = (base_w + c * CB) * L
        pltpu.make_async_copy(comb_v, out_hbm.at[pl.ds(r0, CL)], sem).wait()

    start_gather(0, idx2_a, idx_a, comb_a, sem_a)

    def body2(k, carry):
        c0 = 2 * k

        @pl.when(k > 0)
        def _():
            drain_write(c0 - 1, comb_b, sem_wb)

        start_gather(c0 + 1, idx2_b, idx_b, comb_b, sem_b)
        wait_gather(idx_a, comb_a, sem_a)
        start_write(c0, comb_a, sem_wa)

        @pl.when(k < n2 - 1)
        def _():
            drain_write(c0, comb_a, sem_wa)
            start_gather(c0 + 2, idx2_a, idx_a, comb_a, sem_a)

        wait_gather(idx_b, comb_b, sem_b)
        start_write(c0 + 1, comb_b, sem_wb)
        return carry

    lax.fori_loop(0, n2, body2, 0)
    drain_write(n_chunks - 2, comb_a, sem_wa)
    drain_write(n_chunks - 1, comb_b, sem_wb)


def _comb_body(g_ref, t_ref, o_ref):
    o_ref[:, :DIM] = jnp.tanh(t_ref[...])
    o_ref[:, DIM:OUT_DIM] = g_ref[...]
    o_ref[:, OUT_DIM:] = jnp.zeros(
        (o_ref.shape[0], COMB_DIM - OUT_DIM), jnp.float32)


def _build_comb(glove_table, table):
    v = glove_table.shape[0]
    rows = 2000
    return pl.pallas_call(
        _comb_body,
        grid=(v // rows,),
        in_specs=[pl.BlockSpec((rows, GLOVE_DIM), lambda i: (i, 0)),
                  pl.BlockSpec((rows, DIM), lambda i: (i, 0))],
        out_specs=pl.BlockSpec((rows, COMB_DIM), lambda i: (i, 0)),
        out_shape=jax.ShapeDtypeStruct((v, COMB_DIM), jnp.float32),
    )(glove_table, table)


def _unpad_body(in_ref, o_ref):
    for bi in range(o_ref.shape[0]):
        o_ref[bi] = in_ref[pl.ds(bi * L, L), pl.ds(0, OUT_DIM)]


def _unpad(staged, nb):
    bblk = 8
    return pl.pallas_call(
        _unpad_body,
        grid=(nb // bblk,),
        in_specs=[pl.BlockSpec((bblk * L, COMB_DIM), lambda i: (i, 0))],
        out_specs=pl.BlockSpec((bblk, L, OUT_DIM), lambda i: (i, 0, 0)),
        out_shape=jax.ShapeDtypeStruct((nb, L, OUT_DIM), jnp.float32),
    )(staged)


@jax.jit
def _glove_fused(x, glove_table, table):
    nb = x.shape[0]
    comb = _build_comb(glove_table, table)
    mesh = plsc.VectorSubcoreMesh(
        core_axis_name="c", subcore_axis_name="s",
        num_cores=NC, num_subcores=NS)
    staged = pl.kernel(
        _sc_body,
        out_type=jax.ShapeDtypeStruct((nb * L, COMB_DIM), jnp.float32),
        mesh=mesh,
        scratch_types=[
            pltpu.VMEM((CB, L), jnp.int32),
            pltpu.VMEM((CB, L), jnp.int32),
            pltpu.VMEM((CL,), jnp.int32),
            pltpu.VMEM((CL,), jnp.int32),
            pltpu.VMEM((CL, COMB_DIM), jnp.float32),
            pltpu.VMEM((CL, COMB_DIM), jnp.float32),
            pltpu.SemaphoreType.DMA,
            pltpu.SemaphoreType.DMA,
            pltpu.SemaphoreType.DMA,
            pltpu.SemaphoreType.DMA,
        ],
        compiler_params=pltpu.CompilerParams(
            use_tc_tiling_on_sc=False, needs_layout_passes=False),
    )(x, comb)
    return _unpad(staged, nb)


def kernel(x, glove_table, table):
    return _glove_fused(x, glove_table, table)


# speedup vs baseline: 1.3899x; 1.3899x over previous
import jax
import jax.numpy as jnp
from jax import lax
from jax.experimental import pallas as pl
from jax.experimental.pallas import tpu as pltpu
from jax.experimental.pallas import tpu_sc as plsc

DIM = 128
GLOVE_DIM = 300
OUT_DIM = DIM + GLOVE_DIM
COMB_DIM = 432

NC = 2
NS = 16
NW = NC * NS
LANES = 16

CB = 2
L = 50
CL = CB * L


def _sc_body(x_hbm, comb_hbm, out_hbm,
             idx2_a, idx2_b, idx_a, idx_b, comb_a, comb_b, asm_v,
             sem_a, sem_b, sem_w):
    wid = lax.axis_index("s") * NC + lax.axis_index("c")
    nb_total = x_hbm.shape[0]
    per_w = nb_total // NW
    n_chunks = per_w // CB
    n2 = n_chunks // 2
    base_w = wid * per_w

    def start_gather(c, idx2_v, idx_v, comb_v, sem):
        b0 = base_w + c * CB
        pltpu.sync_copy(x_hbm.at[pl.ds(b0, CB)], idx2_v)

        def flat_body(m, carry2):
            q = jax.lax.iota(jnp.int32, LANES) + m * LANES
            msk = q < CL
            qc = jnp.minimum(q, CL - 1)
            bb = qc // L
            l = qc - bb * L
            vals = plsc.load_gather(idx2_v, [bb, l], mask=msk)
            plsc.store_scatter(idx_v, [qc], vals, mask=msk)
            return carry2

        lax.fori_loop(0, (CL + LANES - 1) // LANES, flat_body, 0, unroll=2)
        pltpu.async_copy(comb_hbm.at[idx_v], comb_v, sem)

    def wait_gather(idx_v, comb_v, sem):
        pltpu.make_async_copy(comb_hbm.at[idx_v], comb_v, sem).wait()

    def drain_write(c):
        b0 = base_w + c * CB
        pltpu.make_async_copy(asm_v, out_hbm.at[pl.ds(b0, CB)], sem_w).wait()

    def assemble_and_write(c, comb_v):
        def row_body(l, carry2):
            for bb in range(CB):
                for j in range(COMB_DIM // LANES):
                    sl = pl.ds(j * LANES, LANES)
                    asm_v[bb, l, sl] = comb_v[bb * L + l, sl]
            return carry2

        lax.fori_loop(0, L, row_body, 0, unroll=2)
        b0 = base_w + c * CB
        pltpu.async_copy(asm_v, out_hbm.at[pl.ds(b0, CB)], sem_w)

    start_gather(0, idx2_a, idx_a, comb_a, sem_a)

    def body2(k, carry):
        c0 = 2 * k
        start_gather(c0 + 1, idx2_b, idx_b, comb_b, sem_b)
        wait_gather(idx_a, comb_a, sem_a)

        @pl.when(k > 0)
        def _():
            drain_write(c0 - 1)

        assemble_and_write(c0, comb_a)

        @pl.when(k < n2 - 1)
        def _():
            start_gather(c0 + 2, idx2_a, idx_a, comb_a, sem_a)

        wait_gather(idx_b, comb_b, sem_b)
        drain_write(c0)
        assemble_and_write(c0 + 1, comb_b)
        return carry

    lax.fori_loop(0, n2, body2, 0)
    drain_write(n_chunks - 1)


def _comb_body(g_ref, t_ref, o_ref):
    o_ref[:, :DIM] = jnp.tanh(t_ref[...])
    o_ref[:, DIM:OUT_DIM] = g_ref[...]
    o_ref[:, OUT_DIM:] = jnp.zeros(
        (o_ref.shape[0], COMB_DIM - OUT_DIM), jnp.float32)


def _build_comb(glove_table, table):
    v = glove_table.shape[0]
    rows = 2000
    return pl.pallas_call(
        _comb_body,
        grid=(v // rows,),
        in_specs=[pl.BlockSpec((rows, GLOVE_DIM), lambda i: (i, 0)),
                  pl.BlockSpec((rows, DIM), lambda i: (i, 0))],
        out_specs=pl.BlockSpec((rows, COMB_DIM), lambda i: (i, 0)),
        out_shape=jax.ShapeDtypeStruct((v, COMB_DIM), jnp.float32),
    )(glove_table, table)


@jax.jit
def _glove_fused(x, glove_table, table):
    nb = x.shape[0]
    comb = _build_comb(glove_table, table)
    mesh = plsc.VectorSubcoreMesh(
        core_axis_name="c", subcore_axis_name="s",
        num_cores=NC, num_subcores=NS)
    staged = pl.kernel(
        _sc_body,
        out_type=jax.ShapeDtypeStruct((nb, L, COMB_DIM), jnp.float32),
        mesh=mesh,
        scratch_types=[
            pltpu.VMEM((CB, L), jnp.int32),
            pltpu.VMEM((CB, L), jnp.int32),
            pltpu.VMEM((CL,), jnp.int32),
            pltpu.VMEM((CL,), jnp.int32),
            pltpu.VMEM((CL, COMB_DIM), jnp.float32),
            pltpu.VMEM((CL, COMB_DIM), jnp.float32),
            pltpu.VMEM((CB, L, COMB_DIM), jnp.float32),
            pltpu.SemaphoreType.DMA,
            pltpu.SemaphoreType.DMA,
            pltpu.SemaphoreType.DMA,
        ],
        compiler_params=pltpu.CompilerParams(
            use_tc_tiling_on_sc=False, needs_layout_passes=False),
    )(x, comb)
    return staged[:, :, :OUT_DIM]


def kernel(x, glove_table, table):
    return _glove_fused(x, glove_table, table)
